# Initial kernel scaffold; baseline (speedup 1.0000x reference)
import jax
import jax.numpy as jnp
from jax.experimental import pallas as pl


def kernel(trg_token_ids_batch, confidence, smoothing_value):
    raise NotImplementedError("write your pallas kernel here")


# speedup vs baseline: 1.8820x; 1.8820x over previous
import jax
import jax.numpy as jnp
from jax.experimental import pallas as pl
from jax.experimental.pallas import tpu as pltpu

_VOCAB = 100000
_PAD_ID = 0
_ROWS_PER_BLOCK = 16


def _fill_kernel(scal_ref, trg_ref, out_ref):
    conf = scal_ref[0]
    base = scal_ref[1]
    trg = trg_ref[...]
    r = trg.shape[0]
    col = jax.lax.broadcasted_iota(jnp.int32, (r, _VOCAB), 1)
    val = jnp.where(col == trg, conf, base)
    val = jnp.where((col == _PAD_ID) | (trg == _PAD_ID), 0.0, val)
    out_ref[...] = val


def kernel(trg_token_ids_batch, confidence, smoothing_value):
    b = trg_token_ids_batch.shape[0]
    base = (smoothing_value / (_VOCAB - 2)).astype(jnp.float32)
    scal = jnp.stack([confidence.astype(jnp.float32), base])
    r = _ROWS_PER_BLOCK
    return pl.pallas_call(
        _fill_kernel,
        grid=(b // r,),
        in_specs=[
            pl.BlockSpec(memory_space=pltpu.SMEM),
            pl.BlockSpec((r, 1), lambda i: (i, 0)),
        ],
        out_specs=pl.BlockSpec((r, _VOCAB), lambda i: (i, 0)),
        out_shape=jax.ShapeDtypeStruct((b, _VOCAB), jnp.float32),
    )(scal, trg_token_ids_batch)


# speedup vs baseline: 1.8857x; 1.0020x over previous
import jax
import jax.numpy as jnp
from jax.experimental import pallas as pl
from jax.experimental.pallas import tpu as pltpu

_VOCAB = 100000
_PAD_ID = 0
_R = 8
_NBUF = 8


def _fill_kernel(scal_ref, trg_ref, out_ref, scratch, sems):
    i = pl.program_id(0)
    n = pl.num_programs(0)
    slot = jax.lax.rem(i, _NBUF)
    conf = scal_ref[0]
    base = scal_ref[1]

    @pl.when(i >= _NBUF)
    def _wait_prev():
        prev = i - _NBUF
        pltpu.make_async_copy(
            scratch.at[slot],
            out_ref.at[pl.ds(prev * _R, _R), :],
            sems.at[slot],
        ).wait()

    trg = trg_ref[pl.ds(i * _R, _R), :]
    col = jax.lax.broadcasted_iota(jnp.int32, (_R, _VOCAB), 1)
    val = jnp.where(col == trg, conf, base)
    val = jnp.where((col == _PAD_ID) | (trg == _PAD_ID), 0.0, val)
    scratch[slot] = val
    pltpu.make_async_copy(
        scratch.at[slot],
        out_ref.at[pl.ds(i * _R, _R), :],
        sems.at[slot],
    ).start()

    @pl.when(i == n - 1)
    def _drain():
        for j in range(_NBUF):
            step = i - (_NBUF - 1) + j
            slot_j = jax.lax.rem(step, _NBUF)
            pltpu.make_async_copy(
                scratch.at[slot_j],
                out_ref.at[pl.ds(step * _R, _R), :],
                sems.at[slot_j],
            ).wait()


def kernel(trg_token_ids_batch, confidence, smoothing_value):
    b = trg_token_ids_batch.shape[0]
    base = (smoothing_value / (_VOCAB - 2)).astype(jnp.float32)
    scal = jnp.stack([confidence.astype(jnp.float32), base])
    return pl.pallas_call(
        _fill_kernel,
        grid=(b // _R,),
        in_specs=[
            pl.BlockSpec(memory_space=pltpu.SMEM),
            pl.BlockSpec((b, 1), lambda i: (0, 0)),
        ],
        out_specs=pl.BlockSpec(memory_space=pl.ANY),
        out_shape=jax.ShapeDtypeStruct((b, _VOCAB), jnp.float32),
        scratch_shapes=[
            pltpu.VMEM((_NBUF, _R, _VOCAB), jnp.float32),
            pltpu.SemaphoreType.DMA((_NBUF,)),
        ],
    )(scal, trg_token_ids_batch)
